# Initial kernel scaffold; baseline (speedup 1.0000x reference)
import jax
import jax.numpy as jnp
from jax.experimental import pallas as pl


def kernel(edges_data, nodes_data, global_data, receivers, senders):
    raise NotImplementedError("write your pallas kernel here")


# speedup vs baseline: 1.7637x; 1.7637x over previous
import functools

import jax
import jax.numpy as jnp
from jax import lax
from jax.experimental import pallas as pl
from jax.experimental.pallas import tpu as pltpu
from jax.experimental.pallas import tpu_sc as plsc

N_NODES = 10000
N_EDGES = 320000
D_FEAT = 128
D_EDGE = 16
D_GLOBAL = 16
D_OUT = D_EDGE + 2 * D_FEAT + D_GLOBAL

_NC = 2
_NS = 16
_NW = _NC * _NS
_E_PER_W = N_EDGES // _NW
_B = 400
_STEPS = _E_PER_W // _B

_mesh = plsc.VectorSubcoreMesh(core_axis_name="c", subcore_axis_name="s")


@functools.partial(
    pl.kernel,
    out_type=jax.ShapeDtypeStruct((N_EDGES, D_OUT), jnp.float32),
    mesh=_mesh,
    compiler_params=pltpu.CompilerParams(use_tc_tiling_on_sc=False),
    scratch_types=[
        pltpu.VMEM((_B,), jnp.int32),
        pltpu.VMEM((_B,), jnp.int32),
        pltpu.VMEM((_B, D_FEAT), jnp.float32),
        pltpu.VMEM((_B, D_FEAT), jnp.float32),
        pltpu.VMEM((_B, D_EDGE), jnp.float32),
        pltpu.VMEM((_B, D_GLOBAL), jnp.float32),
        pltpu.VMEM((D_GLOBAL,), jnp.float32),
        pltpu.SemaphoreType.DMA,
    ],
)
def _edge_block(edges_hbm, nodes_hbm, global_hbm, recv_hbm, send_hbm, out_hbm,
                ridx, sidx, rbuf, sbuf, ebuf, gbuf, gtmp, sem):
    wid = lax.axis_index("s") * _NC + lax.axis_index("c")

    pltpu.sync_copy(global_hbm, gtmp)
    gvec = gtmp[...]

    @pl.loop(0, _B)
    def _fill(i):
        gbuf[i, :] = gvec

    @pl.loop(0, _STEPS)
    def _chunk(step):
        base = wid * _E_PER_W + step * _B
        rows = pl.ds(base, _B)
        pltpu.sync_copy(recv_hbm.at[rows], ridx)
        pltpu.sync_copy(send_hbm.at[rows], sidx)
        pltpu.async_copy(nodes_hbm.at[ridx], rbuf, sem).wait()
        pltpu.async_copy(nodes_hbm.at[sidx], sbuf, sem).wait()
        pltpu.sync_copy(edges_hbm.at[rows], ebuf)
        pltpu.sync_copy(ebuf, out_hbm.at[rows, pl.ds(0, D_EDGE)])
        pltpu.sync_copy(rbuf, out_hbm.at[rows, pl.ds(D_EDGE, D_FEAT)])
        pltpu.sync_copy(sbuf, out_hbm.at[rows, pl.ds(D_EDGE + D_FEAT, D_FEAT)])
        pltpu.sync_copy(gbuf, out_hbm.at[rows, pl.ds(D_EDGE + 2 * D_FEAT, D_GLOBAL)])


def kernel(edges_data, nodes_data, global_data, receivers, senders):
    return _edge_block(
        edges_data,
        nodes_data,
        global_data,
        receivers.astype(jnp.int32),
        senders.astype(jnp.int32),
    )
